# Initial kernel scaffold; baseline (speedup 1.0000x reference)
import functools

import jax
import jax.numpy as jnp
from jax import lax
from jax.experimental import pallas as pl
from jax.experimental.pallas import tpu as pltpu
from jax.experimental.pallas import tpu_sc as plsc

B = 4096
L = 200
D = 32
N_CATE = 1000

NC = 2
NS = 16
NW = NC * NS
BPW = B // NW
HALF = BPW // 2
NFULL = 12
EPI_OFF = L - 16

_mesh = plsc.VectorSubcoreMesh(core_axis_name="c", subcore_axis_name="s")


@functools.partial(
    pl.kernel,
    out_type=[jax.ShapeDtypeStruct((B, D), jnp.float32) for _ in range(5)],
    mesh=_mesh,
    scratch_types=[
        pltpu.VMEM((N_CATE, D), jnp.float32),
        pltpu.VMEM((HALF, L), jnp.int32),
        pltpu.VMEM((HALF, L), jnp.int32),
        pltpu.VMEM((HALF, L), jnp.float32),
        pltpu.VMEM((L, D), jnp.float32),
        pltpu.VMEM((BPW,), jnp.int32),
        pltpu.VMEM((BPW, D), jnp.float32),
        pltpu.VMEM((HALF, D), jnp.float32),
        pltpu.VMEM((HALF, D), jnp.float32),
        pltpu.SemaphoreType.DMA,
    ],
)
def _din_sc_kernel(
    uid_idx, mid_idx, cate_idx, mid_his, cate_his, mask,
    uid_tab, mid_tab, cate_tab,
    out_uid, out_mid, out_cate, out_msum, out_csum,
    cate_v, midx_v, cidx_v, mask_v, rows_v, sidx_v, srow_v, msum_v, csum_v,
    sem,
):
    wid = lax.axis_index("s") * NC + lax.axis_index("c")
    base = wid * BPW
    iota16 = lax.iota(jnp.int32, 16)

    for idx_hbm, tab_hbm, out_hbm in (
        (uid_idx, uid_tab, out_uid),
        (mid_idx, mid_tab, out_mid),
        (cate_idx, cate_tab, out_cate),
    ):
        pltpu.sync_copy(idx_hbm.at[pl.ds(base, BPW)], sidx_v)
        pltpu.async_copy(tab_hbm.at[sidx_v], srow_v, sem).wait()
        pltpu.sync_copy(srow_v, out_hbm.at[pl.ds(base, BPW)])

    pltpu.sync_copy(cate_tab, cate_v)

    def lanes_accum(r, mv, civ, off, js, accs):
        a0, a1, a2, a3 = accs
        for j in js:
            sel = jnp.full((16,), j, dtype=jnp.int32)
            bm = jnp.take(mv, sel, mode="promise_in_bounds")
            ci = jnp.take(civ, sel, mode="promise_in_bounds")
            c0 = plsc.load_gather(cate_v, [ci, iota16])
            c1 = plsc.load_gather(cate_v, [ci, iota16 + 16])
            lrow = off + j
            m0 = rows_v[lrow, pl.ds(0, 16)]
            m1 = rows_v[lrow, pl.ds(16, 16)]
            a0 = a0 + bm * m0
            a1 = a1 + bm * m1
            a2 = a2 + bm * c0
            a3 = a3 + bm * c1
        return (a0, a1, a2, a3)

    def row_body(r, _):
        cp0 = pltpu.async_copy(
            mid_tab.at[midx_v.at[r, pl.ds(0, 128)]],
            rows_v.at[pl.ds(0, 128)], sem)
        cp1 = pltpu.async_copy(
            mid_tab.at[midx_v.at[r, pl.ds(128, L - 128)]],
            rows_v.at[pl.ds(128, L - 128)], sem)
        cp0.wait()
        cp1.wait()

        zero = jnp.zeros((16,), jnp.float32)

        def chunk_body(c, accs):
            off = c * 16
            mv = mask_v[r, pl.ds(off, 16)]
            civ = cidx_v[r, pl.ds(off, 16)]
            return lanes_accum(r, mv, civ, off, range(16), accs)

        accs = lax.fori_loop(0, NFULL, chunk_body, (zero, zero, zero, zero))

        mv = mask_v[r, pl.ds(EPI_OFF, 16)]
        civ = cidx_v[r, pl.ds(EPI_OFF, 16)]
        a0, a1, a2, a3 = lanes_accum(r, mv, civ, EPI_OFF, range(8, 16), accs)

        msum_v[r, pl.ds(0, 16)] = a0
        msum_v[r, pl.ds(16, 16)] = a1
        csum_v[r, pl.ds(0, 16)] = a2
        csum_v[r, pl.ds(16, 16)] = a3
        return 0

    for h in range(2):
        row0 = base + h * HALF
        pltpu.sync_copy(mid_his.at[pl.ds(row0, HALF), :], midx_v)
        pltpu.sync_copy(cate_his.at[pl.ds(row0, HALF), :], cidx_v)
        pltpu.sync_copy(mask.at[pl.ds(row0, HALF), :], mask_v)
        lax.fori_loop(0, HALF, row_body, 0)
        pltpu.sync_copy(msum_v, out_msum.at[pl.ds(row0, HALF)])
        pltpu.sync_copy(csum_v, out_csum.at[pl.ds(row0, HALF)])


def kernel(uid_batch, mid_batch, cate_batch, mid_his_batch, cate_his_batch,
           mask, uid_table, mid_table, cate_table):
    o_uid, o_mid, o_cate, o_msum, o_csum = _din_sc_kernel(
        uid_batch.astype(jnp.int32), mid_batch.astype(jnp.int32),
        cate_batch.astype(jnp.int32), mid_his_batch.astype(jnp.int32),
        cate_his_batch.astype(jnp.int32), mask,
        uid_table, mid_table, cate_table)
    return jnp.concatenate([o_uid, o_mid, o_cate, o_msum, o_csum], axis=1)


# speedup vs baseline: 4.4147x; 4.4147x over previous
import functools

import jax
import jax.numpy as jnp
from jax import lax
from jax.experimental import pallas as pl
from jax.experimental.pallas import tpu as pltpu
from jax.experimental.pallas import tpu_sc as plsc

B = 4096
L = 200
D = 32
N_CATE = 1000

NC = 2
NS = 16
NW = NC * NS
BPW = B // NW
HALF = BPW // 2
NFULL = 12
EPI_OFF = L - 16

_mesh = plsc.VectorSubcoreMesh(core_axis_name="c", subcore_axis_name="s")

_GATHER_DNUMS = lax.GatherDimensionNumbers(
    offset_dims=(), collapsed_slice_dims=(0,), start_index_map=(0,))


def _bcast_lane(v, sel):
    return lax.gather(v, sel[:, None], _GATHER_DNUMS, (1,),
                      mode=lax.GatherScatterMode.PROMISE_IN_BOUNDS)


@functools.partial(
    pl.kernel,
    out_type=[jax.ShapeDtypeStruct((B, D), jnp.float32) for _ in range(5)],
    mesh=_mesh,
    compiler_params=pltpu.CompilerParams(needs_layout_passes=False, use_tc_tiling_on_sc=False),
    scratch_types=[
        pltpu.VMEM((N_CATE, D), jnp.float32),
        pltpu.VMEM((HALF, L), jnp.int32),
        pltpu.VMEM((HALF, L), jnp.int32),
        pltpu.VMEM((HALF, L), jnp.float32),
        pltpu.VMEM((L, D), jnp.float32),
        pltpu.VMEM((BPW,), jnp.int32),
        pltpu.VMEM((BPW, D), jnp.float32),
        pltpu.VMEM((HALF, D), jnp.float32),
        pltpu.VMEM((HALF, D), jnp.float32),
        pltpu.SemaphoreType.DMA,
    ],
)
def _din_sc_kernel(
    uid_idx, mid_idx, cate_idx, mid_his, cate_his, mask,
    uid_tab, mid_tab, cate_tab,
    out_uid, out_mid, out_cate, out_msum, out_csum,
    cate_v, midx_v, cidx_v, mask_v, rows_v, sidx_v, srow_v, msum_v, csum_v,
    sem,
):
    wid = lax.axis_index("s") * NC + lax.axis_index("c")
    base = wid * BPW
    iota16 = lax.iota(jnp.int32, 16)

    for idx_hbm, tab_hbm, out_hbm in (
        (uid_idx, uid_tab, out_uid),
        (mid_idx, mid_tab, out_mid),
        (cate_idx, cate_tab, out_cate),
    ):
        pltpu.sync_copy(idx_hbm.at[pl.ds(base, BPW)], sidx_v)
        pltpu.async_copy(tab_hbm.at[sidx_v], srow_v, sem).wait()
        pltpu.sync_copy(srow_v, out_hbm.at[pl.ds(base, BPW)])

    pltpu.sync_copy(cate_tab, cate_v)

    def lanes_accum(r, mv, civ, off, js, accs):
        a0, a1, a2, a3 = accs
        for j in js:
            sel = jnp.full((16,), j, dtype=jnp.int32)
            bm = _bcast_lane(mv, sel)
            ci = _bcast_lane(civ, sel)
            c0 = plsc.load_gather(cate_v, [ci, iota16])
            c1 = plsc.load_gather(cate_v, [ci, iota16 + 16])
            lrow = off + j
            m0 = rows_v[lrow, pl.ds(0, 16)]
            m1 = rows_v[lrow, pl.ds(16, 16)]
            a0 = a0 + bm * m0
            a1 = a1 + bm * m1
            a2 = a2 + bm * c0
            a3 = a3 + bm * c1
        return (a0, a1, a2, a3)

    def row_body(r, _):
        cp0 = pltpu.async_copy(
            mid_tab.at[midx_v.at[r, pl.ds(0, 128)]],
            rows_v.at[pl.ds(0, 128)], sem)
        cp1 = pltpu.async_copy(
            mid_tab.at[midx_v.at[r, pl.ds(128, L - 128)]],
            rows_v.at[pl.ds(128, L - 128)], sem)
        cp0.wait()
        cp1.wait()

        zero = jnp.zeros((16,), jnp.float32)

        def chunk_body(c, accs):
            off = c * 16
            mv = mask_v[r, pl.ds(off, 16)]
            civ = cidx_v[r, pl.ds(off, 16)]
            return lanes_accum(r, mv, civ, off, range(16), accs)

        accs = lax.fori_loop(0, NFULL, chunk_body, (zero, zero, zero, zero))

        mv = mask_v[r, pl.ds(EPI_OFF, 16)]
        civ = cidx_v[r, pl.ds(EPI_OFF, 16)]
        a0, a1, a2, a3 = lanes_accum(r, mv, civ, EPI_OFF, range(8, 16), accs)

        msum_v[r, pl.ds(0, 16)] = a0
        msum_v[r, pl.ds(16, 16)] = a1
        csum_v[r, pl.ds(0, 16)] = a2
        csum_v[r, pl.ds(16, 16)] = a3
        return 0

    for h in range(2):
        row0 = base + h * HALF
        pltpu.sync_copy(mid_his.at[pl.ds(row0, HALF), :], midx_v)
        pltpu.sync_copy(cate_his.at[pl.ds(row0, HALF), :], cidx_v)
        pltpu.sync_copy(mask.at[pl.ds(row0, HALF), :], mask_v)
        lax.fori_loop(0, HALF, row_body, 0)
        pltpu.sync_copy(msum_v, out_msum.at[pl.ds(row0, HALF)])
        pltpu.sync_copy(csum_v, out_csum.at[pl.ds(row0, HALF)])


def kernel(uid_batch, mid_batch, cate_batch, mid_his_batch, cate_his_batch,
           mask, uid_table, mid_table, cate_table):
    o_uid, o_mid, o_cate, o_msum, o_csum = _din_sc_kernel(
        uid_batch.astype(jnp.int32), mid_batch.astype(jnp.int32),
        cate_batch.astype(jnp.int32), mid_his_batch.astype(jnp.int32),
        cate_his_batch.astype(jnp.int32), mask,
        uid_table, mid_table, cate_table)
    return jnp.concatenate([o_uid, o_mid, o_cate, o_msum, o_csum], axis=1)


# speedup vs baseline: 4.9989x; 1.1323x over previous
import functools

import jax
import jax.numpy as jnp
from jax import lax
from jax.experimental import pallas as pl
from jax.experimental.pallas import tpu as pltpu
from jax.experimental.pallas import tpu_sc as plsc

B = 4096
L = 200
D = 32
N_CATE = 1000

NC = 2
NS = 16
NW = NC * NS
BPW = B // NW
HALF = BPW // 2
NFULL = 12
EPI_OFF = L - 16

_mesh = plsc.VectorSubcoreMesh(core_axis_name="c", subcore_axis_name="s")

_GATHER_DNUMS = lax.GatherDimensionNumbers(
    offset_dims=(), collapsed_slice_dims=(0,), start_index_map=(0,))


def _bcast_lane(v, sel):
    return lax.gather(v, sel[:, None], _GATHER_DNUMS, (1,),
                      mode=lax.GatherScatterMode.PROMISE_IN_BOUNDS)


@functools.partial(
    pl.kernel,
    out_type=[jax.ShapeDtypeStruct((B, D), jnp.float32) for _ in range(5)],
    mesh=_mesh,
    compiler_params=pltpu.CompilerParams(
        needs_layout_passes=False, use_tc_tiling_on_sc=False),
    scratch_types=[
        pltpu.VMEM((N_CATE, D), jnp.float32),
        pltpu.VMEM((HALF, L), jnp.int32),
        pltpu.VMEM((HALF, L), jnp.int32),
        pltpu.VMEM((HALF, L), jnp.float32),
        pltpu.VMEM((L, D), jnp.float32),
        pltpu.VMEM((L, D), jnp.float32),
        pltpu.VMEM((BPW,), jnp.int32),
        pltpu.VMEM((BPW, D), jnp.float32),
        pltpu.VMEM((HALF, D), jnp.float32),
        pltpu.VMEM((HALF, D), jnp.float32),
        pltpu.SemaphoreType.DMA,
        pltpu.SemaphoreType.DMA,
        pltpu.SemaphoreType.DMA,
    ],
)
def _din_sc_kernel(
    uid_idx, mid_idx, cate_idx, mid_his, cate_his, mask,
    uid_tab, mid_tab, cate_tab,
    out_uid, out_mid, out_cate, out_msum, out_csum,
    cate_v, midx_v, cidx_v, mask_v, rows0_v, rows1_v, sidx_v, srow_v,
    msum_v, csum_v, sem, sem0, sem1,
):
    wid = lax.axis_index("s") * NC + lax.axis_index("c")
    base = wid * BPW
    iota16 = lax.iota(jnp.int32, 16)

    for idx_hbm, tab_hbm, out_hbm in (
        (uid_idx, uid_tab, out_uid),
        (mid_idx, mid_tab, out_mid),
        (cate_idx, cate_tab, out_cate),
    ):
        pltpu.sync_copy(idx_hbm.at[pl.ds(base, BPW)], sidx_v)
        pltpu.async_copy(tab_hbm.at[sidx_v], srow_v, sem).wait()
        pltpu.sync_copy(srow_v, out_hbm.at[pl.ds(base, BPW)])

    pltpu.sync_copy(cate_tab, cate_v)

    def row_copies(r, rows_ref, sem_ref):
        return (
            (mid_tab.at[midx_v.at[r, pl.ds(0, 128)]],
             rows_ref.at[pl.ds(0, 128)], sem_ref),
            (mid_tab.at[midx_v.at[r, pl.ds(128, L - 128)]],
             rows_ref.at[pl.ds(128, L - 128)], sem_ref),
        )

    def fire_row(r, rows_ref, sem_ref):
        for src, dst, s in row_copies(r, rows_ref, sem_ref):
            pltpu.async_copy(src, dst, s)

    def wait_row(r, rows_ref, sem_ref):
        for src, dst, s in row_copies(r, rows_ref, sem_ref):
            pltpu.make_async_copy(src, dst, s).wait()

    def lanes_accum(rows_ref, mv, civ, off, js, accs):
        a0, a1, a2, a3 = accs
        for j in js:
            sel = jnp.full((16,), j, dtype=jnp.int32)
            bm = _bcast_lane(mv, sel)
            ci = _bcast_lane(civ, sel)
            c0 = plsc.load_gather(cate_v, [ci, iota16])
            c1 = plsc.load_gather(cate_v, [ci, iota16 + 16])
            lrow = off + j
            m0 = rows_ref[lrow, pl.ds(0, 16)]
            m1 = rows_ref[lrow, pl.ds(16, 16)]
            a0 = a0 + bm * m0
            a1 = a1 + bm * m1
            a2 = a2 + bm * c0
            a3 = a3 + bm * c1
        return (a0, a1, a2, a3)

    zero = jnp.zeros((16,), jnp.float32)

    def compute_row(r, rows_ref):
        def chunk_body(c, accs):
            off = c * 16
            mv = mask_v[r, pl.ds(off, 16)]
            civ = cidx_v[r, pl.ds(off, 16)]
            return lanes_accum(rows_ref, mv, civ, off, range(16), accs)

        accs = lax.fori_loop(0, NFULL, chunk_body, (zero, zero, zero, zero))

        mv = mask_v[r, pl.ds(EPI_OFF, 16)]
        civ = cidx_v[r, pl.ds(EPI_OFF, 16)]
        a0, a1, a2, a3 = lanes_accum(rows_ref, mv, civ, EPI_OFF,
                                     range(8, 16), accs)

        msum_v[r, pl.ds(0, 16)] = a0
        msum_v[r, pl.ds(16, 16)] = a1
        csum_v[r, pl.ds(0, 16)] = a2
        csum_v[r, pl.ds(16, 16)] = a3

    def pair_body(i, _):
        r0 = 2 * i
        r1 = r0 + 1
        fire_row(r1, rows1_v, sem1)
        wait_row(r0, rows0_v, sem0)
        compute_row(r0, rows0_v)

        @pl.when(r0 + 2 < HALF)
        def _():
            fire_row(r0 + 2, rows0_v, sem0)

        wait_row(r1, rows1_v, sem1)
        compute_row(r1, rows1_v)
        return 0

    for h in range(2):
        row0 = base + h * HALF
        pltpu.sync_copy(mid_his.at[pl.ds(row0, HALF), :], midx_v)
        pltpu.sync_copy(cate_his.at[pl.ds(row0, HALF), :], cidx_v)
        pltpu.sync_copy(mask.at[pl.ds(row0, HALF), :], mask_v)
        fire_row(0, rows0_v, sem0)
        lax.fori_loop(0, HALF // 2, pair_body, 0)
        pltpu.sync_copy(msum_v, out_msum.at[pl.ds(row0, HALF)])
        pltpu.sync_copy(csum_v, out_csum.at[pl.ds(row0, HALF)])


def kernel(uid_batch, mid_batch, cate_batch, mid_his_batch, cate_his_batch,
           mask, uid_table, mid_table, cate_table):
    o_uid, o_mid, o_cate, o_msum, o_csum = _din_sc_kernel(
        uid_batch.astype(jnp.int32), mid_batch.astype(jnp.int32),
        cate_batch.astype(jnp.int32), mid_his_batch.astype(jnp.int32),
        cate_his_batch.astype(jnp.int32), mask,
        uid_table, mid_table, cate_table)
    return jnp.concatenate([o_uid, o_mid, o_cate, o_msum, o_csum], axis=1)
